# baseline (speedup 1.0000x reference)
import functools
import math

import jax
import jax.numpy as jnp
from jax import lax
from jax.experimental import pallas as pl
from jax.experimental.pallas import tpu as pltpu
from jax.experimental.pallas import tpu_sc as plsc

_SQRT_2_OVER_PI = math.sqrt(2.0 / math.pi)
_K1 = _SQRT_2_OVER_PI * 0.044715

_SC_ROWS = 1024
_ROWS_PER_W = _SC_ROWS // 32
_CHUNK = 8


def _tc_body(lt_ref, lg_ref, lb_ref, x_ref, p_ref, o_ref):
    del lt_ref, lg_ref, lb_ref, p_ref
    o_ref[:] = x_ref[:] * 1.0001


def _sc_body(x_hbm, o_hbm, buf, sem):
    del sem
    wid = lax.axis_index("s") * 2 + lax.axis_index("c")
    base = wid * _ROWS_PER_W
    for c in range(_ROWS_PER_W // _CHUNK):
        r = base + c * _CHUNK
        pltpu.sync_copy(x_hbm.at[pl.ds(r, _CHUNK)], buf)
        pltpu.sync_copy(buf, o_hbm.at[pl.ds(r, _CHUNK)])


def kernel(x, protos, log_tau, log_gamma, log_blend):
    B, T, D = x.shape
    K = protos.shape[0]
    rows = B * T
    x2 = x.reshape(rows, D)

    x_sc = x2[:_SC_ROWS]
    x_tc = x2[_SC_ROWS:]
    tc_rows = rows - _SC_ROWS

    sc_out = pl.kernel(
        _sc_body,
        mesh=plsc.VectorSubcoreMesh(core_axis_name="c", subcore_axis_name="s"),
        out_type=jax.ShapeDtypeStruct((_SC_ROWS, D), jnp.float32),
        scratch_types=[
            pltpu.VMEM((_CHUNK, D), jnp.float32),
            pltpu.SemaphoreType.DMA,
        ],
    )(x_sc)

    block_rows = 512
    grid = (tc_rows // block_rows,)
    tc_out = pl.pallas_call(
        _tc_body,
        grid=grid,
        in_specs=[
            pl.BlockSpec(memory_space=pltpu.SMEM),
            pl.BlockSpec(memory_space=pltpu.SMEM),
            pl.BlockSpec(memory_space=pltpu.SMEM),
            pl.BlockSpec((block_rows, D), lambda i: (i, 0)),
            pl.BlockSpec((K, D), lambda i: (0, 0)),
        ],
        out_specs=pl.BlockSpec((block_rows, D), lambda i: (i, 0)),
        out_shape=jax.ShapeDtypeStruct((tc_rows, D), x.dtype),
        compiler_params=pltpu.CompilerParams(
            dimension_semantics=("parallel",),
        ),
    )(
        log_tau.reshape(1),
        log_gamma.reshape(1),
        log_blend.reshape(1),
        x_tc,
        protos,
    )
    out = jnp.concatenate([sc_out, tc_out], axis=0)
    return out.reshape(B, T, D)


# speedup vs baseline: 2.9782x; 2.9782x over previous
import math

import jax
import jax.numpy as jnp
from jax.experimental import pallas as pl
from jax.experimental.pallas import tpu as pltpu

_SQRT_2_OVER_PI = math.sqrt(2.0 / math.pi)


_K1 = _SQRT_2_OVER_PI * 0.044715


def _body(lt_ref, lg_ref, lb_ref, x_ref, p_ref, o_ref):
    tau = jnp.exp(lt_ref[0])
    gamma = jnp.exp(lg_ref[0])
    alpha = jax.nn.sigmoid(lb_ref[0])

    xb = x_ref[:]
    x2 = xb * xb
    y = xb * (_K1 * x2 + _SQRT_2_OVER_PI)
    w = xb * (1.0 + jnp.tanh(y))
    w2 = w * w

    p = p_ref[:]
    p_norm = jnp.sqrt(jnp.sum(p * p, axis=-1, keepdims=True))
    pn = p / jnp.maximum(p_norm, 1e-12)

    d = xb.shape[1]
    ones_col = jnp.ones((d, 1), dtype=jnp.float32)
    ssum = jnp.dot(w2, ones_col, preferred_element_type=jnp.float32)
    w_norm = jnp.sqrt(ssum)
    inv_wn = 1.0 / jnp.maximum(w_norm, 2e-12)

    sims = jnp.dot(w, pn.T, preferred_element_type=jnp.float32) * inv_wn

    z = sims * tau
    m = jnp.max(z, axis=-1, keepdims=True)
    lse = m[:, 0] + jnp.log(jnp.sum(jnp.exp(z - m), axis=-1))
    k = p.shape[0]
    soft = (lse - math.log(k)) / tau

    novelty = jnp.exp(-gamma * soft)
    half_gate = 0.5 * (1.0 - alpha + alpha * novelty)
    o_ref[:] = w * half_gate[:, None]


def kernel(x, protos, log_tau, log_gamma, log_blend):
    B, T, D = x.shape
    K = protos.shape[0]
    rows = B * T
    x2 = x.reshape(rows, D)

    block_rows = 512
    grid = (rows // block_rows,)

    out = pl.pallas_call(
        _body,
        grid=grid,
        in_specs=[
            pl.BlockSpec(memory_space=pltpu.SMEM),
            pl.BlockSpec(memory_space=pltpu.SMEM),
            pl.BlockSpec(memory_space=pltpu.SMEM),
            pl.BlockSpec((block_rows, D), lambda i: (i, 0)),
            pl.BlockSpec((K, D), lambda i: (0, 0)),
        ],
        out_specs=pl.BlockSpec((block_rows, D), lambda i: (i, 0)),
        out_shape=jax.ShapeDtypeStruct((rows, D), x.dtype),
        compiler_params=pltpu.CompilerParams(
            dimension_semantics=("parallel",),
        ),
    )(
        log_tau.reshape(1),
        log_gamma.reshape(1),
        log_blend.reshape(1),
        x2,
        protos,
    )
    return out.reshape(B, T, D)


# speedup vs baseline: 2.9816x; 1.0011x over previous
import math

import jax
import jax.numpy as jnp
from jax.experimental import pallas as pl
from jax.experimental.pallas import tpu as pltpu

_SQRT_2_OVER_PI = math.sqrt(2.0 / math.pi)


_K1 = _SQRT_2_OVER_PI * 0.044715


def _body(lt_ref, lg_ref, lb_ref, x_ref, p_ref, o_ref):
    tau = jnp.exp(lt_ref[0])
    gamma = jnp.exp(lg_ref[0])
    alpha = jax.nn.sigmoid(lb_ref[0])

    xb = x_ref[:]
    x2 = xb * xb
    y = xb * (_K1 * x2 + _SQRT_2_OVER_PI)
    w = xb * (1.0 + jnp.tanh(y))
    w2 = w * w

    p = p_ref[:]
    p_norm = jnp.sqrt(jnp.sum(p * p, axis=-1, keepdims=True))
    pn = p / jnp.maximum(p_norm, 1e-12)

    ssum = jnp.sum(w2, axis=-1, keepdims=True)
    w_norm = jnp.sqrt(ssum)
    inv_wn = 1.0 / jnp.maximum(w_norm, 2e-12)

    sims = jnp.dot(w, pn.T, preferred_element_type=jnp.float32) * inv_wn

    z = sims * tau
    m = jnp.max(z, axis=-1, keepdims=True)
    lse = m[:, 0] + jnp.log(jnp.sum(jnp.exp(z - m), axis=-1))
    k = p.shape[0]
    soft = (lse - math.log(k)) / tau

    novelty = jnp.exp(-gamma * soft)
    half_gate = 0.5 * (1.0 - alpha + alpha * novelty)
    o_ref[:] = w * half_gate[:, None]


def kernel(x, protos, log_tau, log_gamma, log_blend):
    B, T, D = x.shape
    K = protos.shape[0]
    rows = B * T
    x2 = x.reshape(rows, D)

    block_rows = 512
    grid = (rows // block_rows,)

    out = pl.pallas_call(
        _body,
        grid=grid,
        in_specs=[
            pl.BlockSpec(memory_space=pltpu.SMEM),
            pl.BlockSpec(memory_space=pltpu.SMEM),
            pl.BlockSpec(memory_space=pltpu.SMEM),
            pl.BlockSpec((block_rows, D), lambda i: (i, 0)),
            pl.BlockSpec((K, D), lambda i: (0, 0)),
        ],
        out_specs=pl.BlockSpec((block_rows, D), lambda i: (i, 0)),
        out_shape=jax.ShapeDtypeStruct((rows, D), x.dtype),
        compiler_params=pltpu.CompilerParams(
            dimension_semantics=("parallel",),
        ),
    )(
        log_tau.reshape(1),
        log_gamma.reshape(1),
        log_blend.reshape(1),
        x2,
        protos,
    )
    return out.reshape(B, T, D)
